# baseline (speedup 1.0000x reference)
import functools

import jax
import jax.numpy as jnp
from jax import lax
from jax.experimental import pallas as pl
from jax.experimental.pallas import tpu as pltpu
from jax.experimental.pallas import tpu_sc as plsc

B = 100000
D = 128
C = 128
NW = 32
NBUF = 6
AHEAD = 3
NCHUNKS = -(-B // C)
LAST_BASE = B - C
CPW = -(-NCHUNKS // NW)
FULL_W = NCHUNKS - (CPW - 1) * NW
ROUNDS = -(-(CPW + AHEAD) // NBUF)


def _sc_gather(idx, table):
    mesh = plsc.VectorSubcoreMesh(core_axis_name="c", subcore_axis_name="s")

    @functools.partial(
        pl.kernel,
        mesh=mesh,
        out_type=jax.ShapeDtypeStruct((B, D), jnp.float32),
        scratch_types=[
            pltpu.VMEM((CPW, C), jnp.int32),
            pltpu.VMEM((NBUF, C, D), jnp.float32),
            pltpu.VMEM_SHARED((16 * 119, D), jnp.float32),
            pltpu.SemaphoreType.DMA,
            *[pltpu.SemaphoreType.DMA] * NBUF,
            *[pltpu.SemaphoreType.DMA] * NBUF,
        ],
    )
    def k(idx_hbm, table_hbm, out_hbm, idx_all, rows, table_v, sem_i,
          g0, g1, g2, g3, g4, g5, s0, s1, s2, s3, s4, s5):
        gsem = [g0, g1, g2, g3, g4, g5]
        ssem = [s0, s1, s2, s3, s4, s5]
        sid = lax.axis_index("s")
        wid = sid * 2 + lax.axis_index("c")
        cnt = (CPW - 1) + (wid < FULL_W).astype(jnp.int32)

        def base_of(j):
            return jnp.minimum((wid + j * NW) * C, LAST_BASE)

        pltpu.sync_copy(table_hbm, table_v.at[pl.ds(sid * 119, 119)])

        def fire_idx(j, carry):
            pltpu.async_copy(idx_hbm.at[pl.ds(base_of(j), C)],
                             idx_all.at[j], sem_i)
            return carry

        def drain_idx(j, carry):
            pltpu.make_async_copy(idx_hbm.at[pl.ds(0, C)],
                                  idx_all.at[j], sem_i).wait()
            return carry

        lax.fori_loop(0, cnt, fire_idx, 0)
        lax.fori_loop(0, cnt, drain_idx, 0)

        def offset_idx(j, carry):
            for u in range(8):
                sl = pl.ds(u * 16, 16)
                idx_all[j, sl] = idx_all[j, sl] + sid * 119
            return carry

        lax.fori_loop(0, cnt, offset_idx, 0)

        def gather(j, b):
            return pltpu.make_async_copy(table_v.at[idx_all.at[j]],
                                         rows.at[b], gsem[b])

        def store(j, b):
            return pltpu.make_async_copy(rows.at[b],
                                         out_hbm.at[pl.ds(base_of(j), C)],
                                         ssem[b])

        for b in range(AHEAD):
            gather(b, b).start()

        def round_body(r, carry):
            for b in range(NBUF):
                j = r * NBUF + b
                bn = (b + AHEAD) % NBUF

                def part_a(j=j, b=b):
                    gather(j, b).wait()
                    store(j, b).start()

                def part_b(j=j, bn=bn):
                    store(j - AHEAD, bn).wait()

                def part_c(j=j, bn=bn):
                    gather(j + AHEAD, bn).start()

                pl.when(j < cnt)(part_a)
                pl.when((j >= AHEAD) & (j - AHEAD < cnt))(part_b)
                pl.when(j + AHEAD < cnt)(part_c)
            return carry

        lax.fori_loop(0, ROUNDS, round_body, 0)

    return k(idx, table)


def kernel(atomic_numbers, table):
    idx = atomic_numbers.astype(jnp.int32)
    return _sc_gather(idx, table)


# speedup vs baseline: 1.0578x; 1.0578x over previous
import functools

import jax
import jax.numpy as jnp
from jax import lax
from jax.experimental import pallas as pl
from jax.experimental.pallas import tpu as pltpu
from jax.experimental.pallas import tpu_sc as plsc

B = 100000
D = 128
C = 128
NW = 32
NBUF = 7
AHEAD = 3
NCHUNKS = -(-B // C)
LAST_BASE = B - C
CPW = -(-NCHUNKS // NW)
FULL_W = NCHUNKS - (CPW - 1) * NW
LAG = NBUF - AHEAD
ROUNDS = -(-(CPW + LAG) // NBUF)


def _sc_gather(idx, table):
    mesh = plsc.VectorSubcoreMesh(core_axis_name="c", subcore_axis_name="s")

    @functools.partial(
        pl.kernel,
        mesh=mesh,
        out_type=jax.ShapeDtypeStruct((B, D), jnp.float32),
        scratch_types=[
            pltpu.VMEM((CPW, C), jnp.int32),
            pltpu.VMEM((NBUF, C, D), jnp.float32),
            pltpu.VMEM_SHARED((119, D), jnp.float32),
            pltpu.SemaphoreType.DMA,
            *[pltpu.SemaphoreType.DMA] * NBUF,
            *[pltpu.SemaphoreType.DMA] * NBUF,
        ],
    )
    def k(idx_hbm, table_hbm, out_hbm, idx_all, rows, table_v, sem_i,
          g0, g1, g2, g3, g4, g5, g6, s0, s1, s2, s3, s4, s5, s6):
        gsem = [g0, g1, g2, g3, g4, g5, g6]
        ssem = [s0, s1, s2, s3, s4, s5, s6]
        wid = lax.axis_index("s") * 2 + lax.axis_index("c")
        cnt = (CPW - 1) + (wid < FULL_W).astype(jnp.int32)

        def base_of(j):
            return jnp.minimum((wid + j * NW) * C, LAST_BASE)

        def fire_idx(j, carry):
            pltpu.async_copy(idx_hbm.at[pl.ds(base_of(j), C)],
                             idx_all.at[j], sem_i)
            return carry

        def drain_idx(j, carry):
            pltpu.make_async_copy(idx_hbm.at[pl.ds(0, C)],
                                  idx_all.at[j], sem_i).wait()
            return carry

        lax.fori_loop(0, cnt, fire_idx, 0)

        pl.when(lax.axis_index("s") == 0)(
            lambda: pltpu.sync_copy(table_hbm, table_v))
        plsc.subcore_barrier()

        lax.fori_loop(0, cnt, drain_idx, 0)

        def gather(j, b):
            return pltpu.make_async_copy(table_v.at[idx_all.at[j]],
                                         rows.at[b], gsem[b])

        def store(j, b):
            return pltpu.make_async_copy(rows.at[b],
                                         out_hbm.at[pl.ds(base_of(j), C)],
                                         ssem[b])

        for b in range(AHEAD):
            gather(b, b).start()

        def round_body(r, carry):
            for b in range(NBUF):
                j = r * NBUF + b
                bn = (b + AHEAD) % NBUF

                def part_a(j=j, b=b):
                    gather(j, b).wait()
                    store(j, b).start()

                def part_b(j=j, bn=bn):
                    store(j - LAG, bn).wait()

                def part_c(j=j, bn=bn):
                    gather(j + AHEAD, bn).start()

                pl.when(j < cnt)(part_a)
                pl.when((j >= LAG) & (j - LAG < cnt))(part_b)
                pl.when(j + AHEAD < cnt)(part_c)
            return carry

        lax.fori_loop(0, ROUNDS, round_body, 0)

    return k(idx, table)


def kernel(atomic_numbers, table):
    idx = atomic_numbers.astype(jnp.int32)
    return _sc_gather(idx, table)
